# Initial kernel scaffold; baseline (speedup 1.0000x reference)
import jax
import jax.numpy as jnp
from jax.experimental import pallas as pl


def kernel(node_latent, edge_index, eW0, eb0, eW1, eb1, eW2, eb2, nW0, nb0, nW1, nb1, nW2, nb2):
    raise NotImplementedError("write your pallas kernel here")


# speedup vs baseline: 3.6109x; 3.6109x over previous
import functools

import jax
import jax.numpy as jnp
from jax import lax
from jax.experimental import pallas as pl
from jax.experimental.pallas import tpu as pltpu
from jax.experimental.pallas import tpu_sc as plsc

H = 256
N_NODES = 10000
N_EDGES = 160000
NC, NS, L = 2, 16, 16
NW = NC * NS
CHUNK = 128
N_CHUNKS = N_EDGES // CHUNK
N_PAD = 10240
ROWS_PER_TILE = N_PAD // NS
ZROWS = 128
HH = H // 2

_SC_MESH = plsc.VectorSubcoreMesh(core_axis_name="c", subcore_axis_name="s",
                                  num_cores=NC, num_subcores=NS)


def _gelu(x):
    return 0.5 * x * (1.0 + lax.erf(x * 0.7071067811865476))


def _bf(x):
    return x.astype(jnp.bfloat16)


def _pq_body(x_ref, w0a_ref, w0b_ref, b0_ref, p_ref, q_ref):
    xb = _bf(x_ref[...])
    p_ref[...] = jnp.dot(xb, _bf(w0a_ref[...]), preferred_element_type=jnp.float32)
    q_ref[...] = (jnp.dot(xb, _bf(w0b_ref[...]), preferred_element_type=jnp.float32)
                  + b0_ref[...])


def _node_pq(x, w0a, w0b, b0):
    blk = 2000
    grid = N_NODES // blk
    return pl.pallas_call(
        _pq_body,
        grid=(grid,),
        in_specs=[
            pl.BlockSpec((blk, H), lambda i: (i, 0)),
            pl.BlockSpec((H, H), lambda i: (0, 0)),
            pl.BlockSpec((H, H), lambda i: (0, 0)),
            pl.BlockSpec((1, H), lambda i: (0, 0)),
        ],
        out_specs=[
            pl.BlockSpec((blk, H), lambda i: (i, 0)),
            pl.BlockSpec((blk, H), lambda i: (i, 0)),
        ],
        out_shape=[
            jax.ShapeDtypeStruct((N_NODES, H), jnp.float32),
            jax.ShapeDtypeStruct((N_NODES, H), jnp.float32),
        ],
    )(x, w0a, w0b, b0)


def _gather_add_body(p_hbm, q_hbm, src_hbm, dst_hbm, u_hbm,
                     sidx, didx, prow, qrow, sem_p, sem_q):
    cid = lax.axis_index("c")
    sid = lax.axis_index("s")
    wid = sid * NC + cid
    lo = (wid * N_CHUNKS) // NW
    hi = ((wid + 1) * N_CHUNKS) // NW

    def chunk_body(c, carry):
        base = c * CHUNK
        pltpu.sync_copy(src_hbm.at[pl.ds(base, CHUNK)], sidx)
        pltpu.sync_copy(dst_hbm.at[pl.ds(base, CHUNK)], didx)
        cp_p = pltpu.async_copy(p_hbm.at[sidx], prow, sem_p)
        cp_q = pltpu.async_copy(q_hbm.at[didx], qrow, sem_q)
        cp_p.wait()
        cp_q.wait()

        def add_row(r, inner):
            for j in range(H // L):
                sl = pl.ds(j * L, L)
                prow[r, sl] = prow[r, sl] + qrow[r, sl]
            return inner

        lax.fori_loop(0, CHUNK, add_row, 0, unroll=False)
        pltpu.sync_copy(prow, u_hbm.at[pl.ds(base, CHUNK)])
        return carry

    lax.fori_loop(lo, hi, chunk_body, 0, unroll=False)


_gather_add = functools.partial(
    pl.kernel,
    out_type=jax.ShapeDtypeStruct((N_EDGES, H), jnp.float32),
    mesh=_SC_MESH,
    scratch_types=[
        pltpu.VMEM((CHUNK,), jnp.int32),
        pltpu.VMEM((CHUNK,), jnp.int32),
        pltpu.VMEM((CHUNK, H), jnp.float32),
        pltpu.VMEM((CHUNK, H), jnp.float32),
        pltpu.SemaphoreType.DMA,
        pltpu.SemaphoreType.DMA,
    ],
)(_gather_add_body)


def _edge_mlp_body(u_ref, w1_ref, b1_ref, w2_ref, b2_ref, m_ref):
    h0 = _gelu(u_ref[...])
    h1 = _gelu(jnp.dot(_bf(h0), _bf(w1_ref[...]),
                       preferred_element_type=jnp.float32) + b1_ref[...])
    m_ref[...] = (jnp.dot(_bf(h1), _bf(w2_ref[...]),
                          preferred_element_type=jnp.float32) + b2_ref[...])


def _edge_mlp(u, w1, b1, w2, b2):
    blk = 2000
    grid = N_EDGES // blk
    return pl.pallas_call(
        _edge_mlp_body,
        grid=(grid,),
        in_specs=[
            pl.BlockSpec((blk, H), lambda i: (i, 0)),
            pl.BlockSpec((H, H), lambda i: (0, 0)),
            pl.BlockSpec((1, H), lambda i: (0, 0)),
            pl.BlockSpec((H, H), lambda i: (0, 0)),
            pl.BlockSpec((1, H), lambda i: (0, 0)),
        ],
        out_specs=pl.BlockSpec((blk, H), lambda i: (i, 0)),
        out_shape=jax.ShapeDtypeStruct((N_EDGES, H), jnp.float32),
    )(u, w1, b1, w2, b2)


def _scatter_add_body(m_hbm, dst_hbm, agg_hbm, idxb, mbuf, zbuf, acc, sem):
    cid = lax.axis_index("c")
    sid = lax.axis_index("s")

    def zero_row(r, carry):
        for j in range(HH // L):
            zbuf[r, pl.ds(j * L, L)] = jnp.zeros((L,), jnp.float32)
        return carry

    lax.fori_loop(0, ZROWS, zero_row, 0, unroll=False)
    for k in range(ROWS_PER_TILE // ZROWS):
        pltpu.sync_copy(zbuf, acc.at[pl.ds(sid * ROWS_PER_TILE + k * ZROWS, ZROWS)])
    plsc.subcore_barrier()

    n_k = jnp.where(sid < N_CHUNKS - NS * (N_CHUNKS // NS),
                    N_CHUNKS // NS + 1, N_CHUNKS // NS)

    def work(col0):
        def chunk_body(k, carry):
            c = k * NS + sid
            base = c * CHUNK
            pltpu.sync_copy(dst_hbm.at[pl.ds(base, CHUNK)], idxb)
            pltpu.sync_copy(m_hbm.at[pl.ds(base, CHUNK), pl.ds(col0, HH)], mbuf)
            pltpu.sync_copy(mbuf, acc.at[idxb], add=True)
            return carry

        lax.fori_loop(0, n_k, chunk_body, 0, unroll=False)
        plsc.subcore_barrier()
        for k in range(ROWS_PER_TILE // ZROWS):
            r0 = sid * ROWS_PER_TILE + k * ZROWS
            pltpu.sync_copy(acc.at[pl.ds(r0, ZROWS)],
                            agg_hbm.at[pl.ds(r0, ZROWS), pl.ds(col0, HH)])

    @pl.when(cid == 0)
    def _():
        work(0)

    @pl.when(cid == 1)
    def _():
        work(HH)


_scatter_add = functools.partial(
    pl.kernel,
    out_type=jax.ShapeDtypeStruct((N_PAD, H), jnp.float32),
    mesh=_SC_MESH,
    scratch_types=[
        pltpu.VMEM((CHUNK,), jnp.int32),
        pltpu.VMEM((CHUNK, HH), jnp.float32),
        pltpu.VMEM((ZROWS, HH), jnp.float32),
        pltpu.VMEM_SHARED((N_PAD, HH), jnp.float32),
        pltpu.SemaphoreType.DMA,
    ],
)(_scatter_add_body)


def _node_mlp_body(x_ref, agg_ref, w0a_ref, w0b_ref, b0_ref, w1_ref, b1_ref,
                   w2_ref, b2_ref, o_ref):
    x = x_ref[...]
    t = (jnp.dot(_bf(x), _bf(w0a_ref[...]), preferred_element_type=jnp.float32)
         + jnp.dot(_bf(agg_ref[...]), _bf(w0b_ref[...]),
                   preferred_element_type=jnp.float32)
         + b0_ref[...])
    h = _gelu(t)
    h = _gelu(jnp.dot(_bf(h), _bf(w1_ref[...]),
                      preferred_element_type=jnp.float32) + b1_ref[...])
    o_ref[...] = (x + jnp.dot(_bf(h), _bf(w2_ref[...]),
                              preferred_element_type=jnp.float32) + b2_ref[...])


def _node_mlp(x, agg, w0a, w0b, b0, w1, b1, w2, b2):
    blk = 2000
    grid = N_NODES // blk
    wspec = pl.BlockSpec((H, H), lambda i: (0, 0))
    bspec = pl.BlockSpec((1, H), lambda i: (0, 0))
    return pl.pallas_call(
        _node_mlp_body,
        grid=(grid,),
        in_specs=[
            pl.BlockSpec((blk, H), lambda i: (i, 0)),
            pl.BlockSpec((blk, H), lambda i: (i, 0)),
            wspec, wspec, bspec, wspec, bspec, wspec, bspec,
        ],
        out_specs=pl.BlockSpec((blk, H), lambda i: (i, 0)),
        out_shape=jax.ShapeDtypeStruct((N_NODES, H), jnp.float32),
    )(x, agg, w0a, w0b, b0, w1, b1, w2, b2)


def kernel(node_latent, edge_index, eW0, eb0, eW1, eb1, eW2, eb2,
           nW0, nb0, nW1, nb1, nW2, nb2):
    src = edge_index[0].astype(jnp.int32)
    dst = edge_index[1].astype(jnp.int32)

    p, q = _node_pq(node_latent, eW0[:H], eW0[H:], eb0.reshape(1, H))
    u = _gather_add(p, q, src, dst)
    m = _edge_mlp(u, eW1, eb1.reshape(1, H), eW2, eb2.reshape(1, H))
    agg = _scatter_add(m, dst)
    return _node_mlp(node_latent, agg, nW0[:H], nW0[H:], nb0.reshape(1, H),
                     nW1, nb1.reshape(1, H), nW2, nb2.reshape(1, H))


# speedup vs baseline: 5.8322x; 1.6152x over previous
import functools

import jax
import jax.numpy as jnp
from jax import lax
from jax.experimental import pallas as pl
from jax.experimental.pallas import tpu as pltpu
from jax.experimental.pallas import tpu_sc as plsc

H = 256
N_NODES = 10000
N_EDGES = 160000
NC, NS, L = 2, 16, 16
NW = NC * NS
CHUNK = 128
N_CHUNKS = N_EDGES // CHUNK
N_PAD = 10112
ROWS_PER_TILE = N_PAD // NS
ZROWS = 128
ZSTEPS = (128, 128, 128, 128, 120)
HH = H // 2

_SC_MESH = plsc.VectorSubcoreMesh(core_axis_name="c", subcore_axis_name="s",
                                  num_cores=NC, num_subcores=NS)


def _gelu(x):
    return 0.5 * x * (1.0 + lax.erf(x * 0.7071067811865476))


def _bf(x):
    return x.astype(jnp.bfloat16)


def _bf16_bits_hi(x):
    b = jax.lax.bitcast_convert_type(x, jnp.int32)
    r = b + jnp.int32(0x7FFF) + ((b >> 16) & 1)
    return r & jnp.int32(-65536)


def _pack_pairs(p):
    lo = jax.lax.shift_right_logical(_bf16_bits_hi(p[:, :HH]), 16)
    return lo | _bf16_bits_hi(p[:, HH:])


def _unpack_pairs(w):
    lo = jax.lax.bitcast_convert_type(w << 16, jnp.float32)
    hi = jax.lax.bitcast_convert_type(w & jnp.int32(-65536), jnp.float32)
    return jnp.concatenate([lo, hi], axis=1)


def _pq_body(x_ref, w0a_ref, w0b_ref, b0_ref, p_ref, q_ref):
    xb = _bf(x_ref[...])
    p_ref[...] = _pack_pairs(jnp.dot(xb, _bf(w0a_ref[...]),
                                     preferred_element_type=jnp.float32))
    q_ref[...] = _pack_pairs(jnp.dot(xb, _bf(w0b_ref[...]),
                                     preferred_element_type=jnp.float32)
                             + b0_ref[...])


def _node_pq(x, w0a, w0b, b0):
    blk = 2000
    grid = N_NODES // blk
    return pl.pallas_call(
        _pq_body,
        grid=(grid,),
        in_specs=[
            pl.BlockSpec((blk, H), lambda i: (i, 0)),
            pl.BlockSpec((H, H), lambda i: (0, 0)),
            pl.BlockSpec((H, H), lambda i: (0, 0)),
            pl.BlockSpec((1, H), lambda i: (0, 0)),
        ],
        out_specs=[
            pl.BlockSpec((blk, HH), lambda i: (i, 0)),
            pl.BlockSpec((blk, HH), lambda i: (i, 0)),
        ],
        out_shape=[
            jax.ShapeDtypeStruct((N_NODES, HH), jnp.int32),
            jax.ShapeDtypeStruct((N_NODES, HH), jnp.int32),
        ],
    )(x, w0a, w0b, b0)


def _gather_add_body(p_hbm, q_hbm, src_hbm, dst_hbm, us_hbm, ud_hbm,
                     sidx, didx, prow, qrow, psem, qsem, ossem, odsem):
    cid = lax.axis_index("c")
    sid = lax.axis_index("s")
    wid = sid * NC + cid
    lo = (wid * N_CHUNKS) // NW
    hi = ((wid + 1) * N_CHUNKS) // NW

    def issue(c):
        b = (c - lo) & 1
        base = c * CHUNK

        @pl.when(c - lo >= 2)
        def _():
            pltpu.make_async_copy(prow.at[b], us_hbm.at[pl.ds(base, CHUNK)],
                                  ossem.at[b]).wait()
            pltpu.make_async_copy(qrow.at[b], ud_hbm.at[pl.ds(base, CHUNK)],
                                  odsem.at[b]).wait()

        pltpu.sync_copy(src_hbm.at[pl.ds(base, CHUNK)], sidx.at[b])
        pltpu.sync_copy(dst_hbm.at[pl.ds(base, CHUNK)], didx.at[b])
        pltpu.async_copy(p_hbm.at[sidx.at[b]], prow.at[b], psem.at[b])
        pltpu.async_copy(q_hbm.at[didx.at[b]], qrow.at[b], qsem.at[b])

    issue(lo)

    def chunk_body(c, carry):
        b = (c - lo) & 1
        base = c * CHUNK

        @pl.when(c + 1 < hi)
        def _():
            issue(c + 1)

        pltpu.make_async_copy(p_hbm.at[sidx.at[b]], prow.at[b],
                              psem.at[b]).wait()
        pltpu.make_async_copy(q_hbm.at[didx.at[b]], qrow.at[b],
                              qsem.at[b]).wait()
        pltpu.async_copy(prow.at[b], us_hbm.at[pl.ds(base, CHUNK)],
                         ossem.at[b])
        pltpu.async_copy(qrow.at[b], ud_hbm.at[pl.ds(base, CHUNK)],
                         odsem.at[b])
        return carry

    lax.fori_loop(lo, hi, chunk_body, 0, unroll=False)

    for back in (1, 2):
        b = (hi - back - lo) & 1
        base = (hi - back) * CHUNK
        pltpu.make_async_copy(prow.at[b], us_hbm.at[pl.ds(base, CHUNK)],
                              ossem.at[b]).wait()
        pltpu.make_async_copy(qrow.at[b], ud_hbm.at[pl.ds(base, CHUNK)],
                              odsem.at[b]).wait()


_gather_add = functools.partial(
    pl.kernel,
    out_type=[
        jax.ShapeDtypeStruct((N_EDGES, HH), jnp.int32),
        jax.ShapeDtypeStruct((N_EDGES, HH), jnp.int32),
    ],
    mesh=_SC_MESH,
    scratch_types=[
        pltpu.VMEM((2, CHUNK), jnp.int32),
        pltpu.VMEM((2, CHUNK), jnp.int32),
        pltpu.VMEM((2, CHUNK, HH), jnp.int32),
        pltpu.VMEM((2, CHUNK, HH), jnp.int32),
        pltpu.SemaphoreType.DMA((2,)),
        pltpu.SemaphoreType.DMA((2,)),
        pltpu.SemaphoreType.DMA((2,)),
        pltpu.SemaphoreType.DMA((2,)),
    ],
)(_gather_add_body)


def _edge_mlp_body(us_ref, ud_ref, w1_ref, b1_ref, w2_ref, b2_ref, m_ref):
    h0 = _gelu(_unpack_pairs(us_ref[...]) + _unpack_pairs(ud_ref[...]))
    h1 = _gelu(jnp.dot(_bf(h0), _bf(w1_ref[...]),
                       preferred_element_type=jnp.float32) + b1_ref[...])
    m_ref[...] = (jnp.dot(_bf(h1), _bf(w2_ref[...]),
                          preferred_element_type=jnp.float32) + b2_ref[...])


def _edge_mlp(us, ud, w1, b1, w2, b2):
    blk = 2000
    grid = N_EDGES // blk
    return pl.pallas_call(
        _edge_mlp_body,
        grid=(grid,),
        in_specs=[
            pl.BlockSpec((blk, HH), lambda i: (i, 0)),
            pl.BlockSpec((blk, HH), lambda i: (i, 0)),
            pl.BlockSpec((H, H), lambda i: (0, 0)),
            pl.BlockSpec((1, H), lambda i: (0, 0)),
            pl.BlockSpec((H, H), lambda i: (0, 0)),
            pl.BlockSpec((1, H), lambda i: (0, 0)),
        ],
        out_specs=pl.BlockSpec((blk, H), lambda i: (i, 0)),
        out_shape=jax.ShapeDtypeStruct((N_EDGES, H), jnp.float32),
    )(us, ud, w1, b1, w2, b2)


def _scatter_add_body(m_hbm, dst_hbm, agg_hbm, idxb, mbuf, zbuf, acc, msem):
    cid = lax.axis_index("c")
    sid = lax.axis_index("s")

    def zero_row(r, carry):
        for j in range(HH // L):
            zbuf[r, pl.ds(j * L, L)] = jnp.zeros((L,), jnp.float32)
        return carry

    lax.fori_loop(0, ZROWS, zero_row, 0, unroll=False)
    off = 0
    for n in ZSTEPS:
        pltpu.sync_copy(zbuf.at[pl.ds(0, n)],
                        acc.at[pl.ds(sid * ROWS_PER_TILE + off, n)])
        off += n
    plsc.subcore_barrier()

    n_k = jnp.where(sid < N_CHUNKS - NS * (N_CHUNKS // NS),
                    N_CHUNKS // NS + 1, N_CHUNKS // NS)

    def work(col0):
        def issue(k):
            b = k & 1
            base = (k * NS + sid) * CHUNK
            pltpu.sync_copy(dst_hbm.at[pl.ds(base, CHUNK)], idxb.at[b])
            pltpu.async_copy(m_hbm.at[pl.ds(base, CHUNK), pl.ds(col0, HH)],
                             mbuf.at[b], msem.at[b])

        issue(0)

        def chunk_body(k, carry):
            b = k & 1
            base = (k * NS + sid) * CHUNK

            @pl.when(k + 1 < n_k)
            def _():
                issue(k + 1)

            pltpu.make_async_copy(
                m_hbm.at[pl.ds(base, CHUNK), pl.ds(col0, HH)],
                mbuf.at[b], msem.at[b]).wait()
            pltpu.sync_copy(mbuf.at[b], acc.at[idxb.at[b]], add=True)
            return carry

        lax.fori_loop(0, n_k, chunk_body, 0, unroll=False)
        plsc.subcore_barrier()
        off = 0
        for n in ZSTEPS:
            r0 = sid * ROWS_PER_TILE + off
            pltpu.sync_copy(acc.at[pl.ds(r0, n)],
                            agg_hbm.at[pl.ds(r0, n), pl.ds(col0, HH)])
            off += n

    @pl.when(cid == 0)
    def _():
        work(0)

    @pl.when(cid == 1)
    def _():
        work(HH)


_scatter_add = functools.partial(
    pl.kernel,
    out_type=jax.ShapeDtypeStruct((N_PAD, H), jnp.float32),
    mesh=_SC_MESH,
    scratch_types=[
        pltpu.VMEM((2, CHUNK), jnp.int32),
        pltpu.VMEM((2, CHUNK, HH), jnp.float32),
        pltpu.VMEM((ZROWS, HH), jnp.float32),
        pltpu.VMEM_SHARED((N_PAD, HH), jnp.float32),
        pltpu.SemaphoreType.DMA((2,)),
    ],
)(_scatter_add_body)


def _node_mlp_body(x_ref, agg_ref, w0a_ref, w0b_ref, b0_ref, w1_ref, b1_ref,
                   w2_ref, b2_ref, o_ref):
    x = x_ref[...]
    t = (jnp.dot(_bf(x), _bf(w0a_ref[...]), preferred_element_type=jnp.float32)
         + jnp.dot(_bf(agg_ref[...]), _bf(w0b_ref[...]),
                   preferred_element_type=jnp.float32)
         + b0_ref[...])
    h = _gelu(t)
    h = _gelu(jnp.dot(_bf(h), _bf(w1_ref[...]),
                      preferred_element_type=jnp.float32) + b1_ref[...])
    o_ref[...] = (x + jnp.dot(_bf(h), _bf(w2_ref[...]),
                              preferred_element_type=jnp.float32) + b2_ref[...])


def _node_mlp(x, agg, w0a, w0b, b0, w1, b1, w2, b2):
    blk = 2000
    grid = N_NODES // blk
    wspec = pl.BlockSpec((H, H), lambda i: (0, 0))
    bspec = pl.BlockSpec((1, H), lambda i: (0, 0))
    return pl.pallas_call(
        _node_mlp_body,
        grid=(grid,),
        in_specs=[
            pl.BlockSpec((blk, H), lambda i: (i, 0)),
            pl.BlockSpec((blk, H), lambda i: (i, 0)),
            wspec, wspec, bspec, wspec, bspec, wspec, bspec,
        ],
        out_specs=pl.BlockSpec((blk, H), lambda i: (i, 0)),
        out_shape=jax.ShapeDtypeStruct((N_NODES, H), jnp.float32),
    )(x, agg, w0a, w0b, b0, w1, b1, w2, b2)


def kernel(node_latent, edge_index, eW0, eb0, eW1, eb1, eW2, eb2,
           nW0, nb0, nW1, nb1, nW2, nb2):
    src = edge_index[0].astype(jnp.int32)
    dst = edge_index[1].astype(jnp.int32)

    p, q = _node_pq(node_latent, eW0[:H], eW0[H:], eb0.reshape(1, H))
    us, ud = _gather_add(p, q, src, dst)
    m = _edge_mlp(us, ud, eW1, eb1.reshape(1, H), eW2, eb2.reshape(1, H))
    agg = _scatter_add(m, dst)
    return _node_mlp(node_latent, agg, nW0[:H], nW0[H:], nb0.reshape(1, H),
                     nW1, nb1.reshape(1, H), nW2, nb2.reshape(1, H))
